# baseline (speedup 1.0000x reference)
import functools

import jax
import jax.numpy as jnp
from jax import lax
from jax.experimental import pallas as pl
from jax.experimental.pallas import tpu as pltpu
from jax.experimental.pallas import tpu_sc as plsc

_K = 256
_R = 128
_N = 32768
_NW = 32
_RPW = _R // _NW
_NCHUNK = _N // 16

_NB1 = 2048
_NB2 = 2048
_NB3 = 1024
_CAP = 1024


def _lane():
    return lax.iota(jnp.int32, 16)


def _monotonic_u32(v):
    u = plsc.bitcast(v, jnp.uint32)
    flip = jnp.where(
        u >= jnp.uint32(0x80000000), jnp.uint32(0xFFFFFFFF), jnp.uint32(0x80000000)
    )
    return u ^ flip


def _vsort_kv(k, v, descending):
    return plsc.sort_key_val(k, v, descending=descending)


def _rev16(x):
    return lax.rev(x, (0,))


def _ce_kv(ak, av, bk, bv, descending):
    m = (ak >= bk) if descending else (ak <= bk)
    return (
        jnp.where(m, ak, bk), jnp.where(m, av, bv),
        jnp.where(m, bk, ak), jnp.where(m, bv, av),
    )


def _sort16(ks, vs, descending):
    ks = list(ks)
    vs = list(vs)
    for i in range(16):
        ks[i], vs[i] = _vsort_kv(ks[i], vs[i], descending)
    size = 2
    while size <= 16:
        half = size // 2
        for base in range(0, 16, size):
            sub_k = ks[base:base + half] + \
                [_rev16(k) for k in ks[base + half:base + size]][::-1]
            sub_v = vs[base:base + half] + \
                [_rev16(v) for v in vs[base + half:base + size]][::-1]
            d = half
            while d >= 1:
                for j in range(size):
                    if (j % (2 * d)) < d:
                        lk, lv, hk, hv = _ce_kv(sub_k[j], sub_v[j],
                                                sub_k[j + d], sub_v[j + d],
                                                descending)
                        sub_k[j], sub_v[j] = lk, lv
                        sub_k[j + d], sub_v[j + d] = hk, hv
                d //= 2
            for j in range(size):
                sub_k[j], sub_v[j] = _vsort_kv(sub_k[j], sub_v[j], descending)
            ks[base:base + size] = sub_k
            vs[base:base + size] = sub_v
        size *= 2
    return ks, vs


def _suffix_select(hist_ref, nb, tots_ref, ts_ref, kp):
    nch = nb // 16
    lane0 = _lane() == 0

    def _tbody(j4, _):
        base = j4 * 4
        vs = [hist_ref[pl.ds((base + u) * 16, 16)] for u in range(4)]
        tots = [jnp.sum(v) for v in vs]
        for u in range(4):
            plsc.store_scatter(
                tots_ref, [jnp.full((16,), base + u, jnp.int32)],
                jnp.broadcast_to(tots[u], (16,)), mask=lane0,
            )
        return 0

    lax.fori_loop(0, nch // 4, _tbody, 0)

    nv = nch // 16
    vs = [tots_ref[pl.ds(jv * 16, 16)] for jv in range(nv)]
    css = [lax.rev(plsc.cumsum(lax.rev(v, (0,))), (0,)) for v in vs]
    tl = [jnp.max(cs) for cs in css]
    run = jnp.int32(0)
    for jv in range(nv - 1, -1, -1):
        sfx = css[jv] + run
        ts_ref[pl.ds(jv * 16, 16)] = sfx
        run = run + tl[jv]

    cnt = jnp.zeros((16,), jnp.int32)
    for jv in range(nch // 16):
        v = ts_ref[pl.ds(jv * 16, 16)]
        cnt = cnt + jnp.where(v >= kp, 1, 0)
    j_sel = jnp.sum(cnt) - 1

    nxt = jnp.minimum(j_sel + 1, nch - 1)
    above_chunks = jnp.max(plsc.load_gather(ts_ref, [jnp.full((16,), nxt, jnp.int32)]))
    above_chunks = jnp.where(j_sel == nch - 1, 0, above_chunks)

    s_chunk = hist_ref[pl.ds(j_sel * 16, 16)]
    sfx_in = lax.rev(plsc.cumsum(lax.rev(s_chunk, (0,))), (0,)) + above_chunks
    b_local = jnp.sum(jnp.where(sfx_in >= kp, 1, 0)) - 1
    bucket = j_sel * 16 + b_local
    above = above_chunks + jnp.sum(jnp.where(_lane() > b_local, s_chunk, 0))
    inside = jnp.sum(jnp.where(_lane() == b_local, s_chunk, 0))
    return bucket, above, inside


def _topk_body(x_hbm, out_hbm, buf, hist1, hist2, hist3, tots, ts,
               sub_key, sub_pos, gt_key, gt_idx, eq_idx,
               cand_key, cand_idx, outrow):
    wid = lax.axis_index("s") * 2 + lax.axis_index("c")
    ones = jnp.ones((16,), jnp.int32)
    zeros16 = jnp.zeros((16,), jnp.int32)
    lane = _lane()

    def row_body(r, _):
        row = wid * _RPW + r

        pltpu.sync_copy(x_hbm.at[row], buf)

        def _zbody(i8, _):
            for u in range(8):
                i = i8 * 8 + u
                hist1[pl.ds(i * 16, 16)] = zeros16
                hist2[pl.ds(i * 16, 16)] = zeros16

                @pl.when(i < _NB3 // 16)
                def _():
                    hist3[pl.ds(i * 16, 16)] = zeros16

            return 0

        lax.fori_loop(0, _NB1 // 128, _zbody, 0)

        def _p0(i8, _):
            base = i8 * 8
            vs = [buf[pl.ds((base + u) * 16, 16)] for u in range(8)]
            mus = [_monotonic_u32(v) for v in vs]
            b1s = [lax.convert_element_type(mu >> 21, jnp.int32) for mu in mus]
            for u in range(8):
                plsc.addupdate_scatter(hist1, [b1s[u]], ones)
            return 0

        lax.fori_loop(0, _NCHUNK // 8, _p0, 0)

        bkt1, above1, s1 = _suffix_select(hist1, _NB1, tots, ts, jnp.int32(_K))
        k2 = jnp.int32(_K) - above1
        b1u = lax.convert_element_type(bkt1, jnp.uint32)
        csub = above1 + s1

        def _refine(mu_of, pos_of, nchunks, valid_of):
            ng = (nchunks + 3) >> 2

            def _p2(g, _):
                base = g * 4
                mus = [mu_of(base + u) for u in range(4)]
                ms = [valid_of(base + u) & ((mus[u] >> 21) == b1u)
                      for u in range(4)]
                b2s = [lax.convert_element_type(
                    (mu >> 10) & jnp.uint32(0x7FF), jnp.int32) for mu in mus]
                for u in range(4):
                    plsc.addupdate_scatter(hist2, [b2s[u]], ones, mask=ms[u])
                return 0

            lax.fori_loop(0, ng, _p2, 0)
            bkt2, above2, _ = _suffix_select(hist2, _NB2, tots, ts, k2)
            k3 = k2 - above2
            pre2 = (b1u << 11) | lax.convert_element_type(bkt2, jnp.uint32)

            def _p3(g, _):
                base = g * 4
                mus = [mu_of(base + u) for u in range(4)]
                ms = [valid_of(base + u) & ((mus[u] >> 10) == pre2)
                      for u in range(4)]
                b3s = [lax.convert_element_type(mu & jnp.uint32(0x3FF),
                                                jnp.int32) for mu in mus]
                for u in range(4):
                    plsc.addupdate_scatter(hist3, [b3s[u]], ones, mask=ms[u])
                return 0

            lax.fori_loop(0, ng, _p3, 0)
            bkt3, above3, _ = _suffix_select(hist3, _NB3, tots, ts, k3)
            need = k3 - above3
            c_gt = jnp.int32(_K) - need
            t_u = (pre2 << 10) | lax.convert_element_type(bkt3, jnp.uint32)
            t_splat = jnp.broadcast_to(t_u, (16,))

            def _cbody(g, carry):
                off_gt, off_eq = carry
                base = g * 4
                mus = [mu_of(base + u) for u in range(4)]
                poss = [pos_of(base + u) for u in range(4)]
                m_gts = [valid_of(base + u) & (mus[u] > t_splat)
                         for u in range(4)]
                m_eqs = [valid_of(base + u) & (mus[u] == t_splat)
                         for u in range(4)]
                cs_gts = [plsc.cumsum(ones, mask=m) for m in m_gts]
                cs_eqs = [plsc.cumsum(ones, mask=m) for m in m_eqs]
                n_gts = [plsc.all_reduce_population_count(m) for m in m_gts]
                n_eqs = [plsc.all_reduce_population_count(m) for m in m_eqs]
                for u in range(4):
                    p_gt = off_gt + cs_gts[u] - 1
                    p_eq = off_eq + cs_eqs[u] - 1
                    plsc.store_scatter(gt_key, [p_gt],
                                       plsc.bitcast(mus[u], jnp.int32),
                                       mask=m_gts[u])
                    plsc.store_scatter(gt_idx, [p_gt], poss[u], mask=m_gts[u])
                    plsc.store_scatter(eq_idx, [p_eq], poss[u],
                                       mask=m_eqs[u] & (p_eq < need))
                    off_gt = off_gt + n_gts[u]
                    off_eq = off_eq + n_eqs[u]
                return off_gt, off_eq

            lax.fori_loop(0, ng, _cbody, (zeros16, zeros16))
            return t_splat, need, c_gt

        def _main_path(_):
            def _collect(i8, off):
                base = i8 * 8
                mus = [_monotonic_u32(buf[pl.ds((base + u) * 16, 16)])
                       for u in range(8)]
                ms = [(mu >> 21) >= b1u for mu in mus]
                css = [plsc.cumsum(ones, mask=m) for m in ms]
                cnts = [plsc.all_reduce_population_count(m) for m in ms]
                for u in range(8):
                    p = off + css[u] - 1
                    plsc.store_scatter(sub_key, [p],
                                       plsc.bitcast(mus[u], jnp.int32),
                                       mask=ms[u])
                    plsc.store_scatter(sub_pos, [p], (base + u) * 16 + lane,
                                       mask=ms[u])
                    off = off + cnts[u]
                return off

            lax.fori_loop(0, _NCHUNK // 8, _collect, zeros16)

            csub_splat = jnp.broadcast_to(csub, (16,))
            nsub = (csub + 15) >> 4

            def pos_of(i):
                return sub_pos[pl.ds(i * 16, 16)]

            def mu_of(i):
                return plsc.bitcast(sub_key[pl.ds(i * 16, 16)], jnp.uint32)

            def valid_of(i):
                return (i * 16 + lane) < csub_splat

            return _refine(mu_of, pos_of, nsub, valid_of)

        def _fallback(_):
            def mu_of(i):
                return _monotonic_u32(buf[pl.ds(i * 16, 16)])

            def pos_of(i):
                return i * 16 + lane

            def valid_of(i):
                return lane >= 0

            return _refine(mu_of, pos_of, _NCHUNK, valid_of)

        t_splat, need, c_gt = lax.cond(csub <= _CAP, _main_path, _fallback, 0)

        ks = []
        ps = []
        for t in range(_K // 16):
            j = t * 16 + lane
            from_gt = j < c_gt
            jg = jnp.where(from_gt, j, 0)
            je = jnp.maximum(j - c_gt, 0)
            kk = plsc.bitcast(plsc.load_gather(gt_key, [jg]), jnp.uint32)
            key_t = jnp.where(from_gt, kk, t_splat)
            idx_t = jnp.where(
                from_gt,
                plsc.load_gather(gt_idx, [jg]),
                plsc.load_gather(eq_idx, [je]),
            )
            cand_idx[pl.ds(t * 16, 16)] = idx_t
            ks.append(key_t)
            ps.append(j)

        ks, ps = _sort16(ks, ps, descending=True)
        for t in range(_K // 16):
            cand_key[pl.ds(t * 16, 16)] = plsc.bitcast(ks[t], jnp.int32)

        sts = []
        for t in range(_K // 16):
            qidx = t * 16 + lane
            sh = plsc.bitcast(
                plsc.load_gather(cand_key, [jnp.maximum(qidx - 1, 0)]),
                jnp.uint32,
            )
            d = ks[t] != sh
            if t == 0:
                d = d | (lane == 0)
            sts.append(jnp.where(d, qidx, 0))
        comp = []
        carry = jnp.int32(0)
        for t in range(_K // 16):
            rs = jnp.maximum(plsc.cummax(sts[t]), carry)
            comp.append((rs << 8) + ps[t])
            carry = jnp.maximum(carry, jnp.max(sts[t]))
        _, ps2 = _sort16(comp, ps, descending=False)

        for t in range(_K // 16):
            outrow[pl.ds(t * 16, 16)] = plsc.load_gather(cand_idx, [ps2[t]])

        pltpu.sync_copy(outrow, out_hbm.at[row])
        return 0

    lax.fori_loop(0, _RPW, row_body, 0)


@jax.jit
def kernel(x):
    mesh = plsc.VectorSubcoreMesh(core_axis_name="c", subcore_axis_name="s")
    f = functools.partial(
        pl.kernel,
        mesh=mesh,
        compiler_params=pltpu.CompilerParams(needs_layout_passes=False),
        out_type=jax.ShapeDtypeStruct((_R, _K), jnp.int32),
        scratch_types=[
            pltpu.VMEM((_N,), jnp.float32),
            pltpu.VMEM((_NB1,), jnp.int32),
            pltpu.VMEM((_NB2,), jnp.int32),
            pltpu.VMEM((_NB3,), jnp.int32),
            pltpu.VMEM((256,), jnp.int32),
            pltpu.VMEM((256,), jnp.int32),
            pltpu.VMEM((_CAP + 16,), jnp.int32),
            pltpu.VMEM((_CAP + 16,), jnp.int32),
            pltpu.VMEM((512,), jnp.int32),
            pltpu.VMEM((512,), jnp.int32),
            pltpu.VMEM((512,), jnp.int32),
            pltpu.VMEM((_K,), jnp.int32),
            pltpu.VMEM((_K,), jnp.int32),
            pltpu.VMEM((_K,), jnp.int32),
        ],
    )(_topk_body)
    return f(x)


# speedup vs baseline: 1.0986x; 1.0986x over previous
import functools

import jax
import jax.numpy as jnp
from jax import lax
from jax.experimental import pallas as pl
from jax.experimental.pallas import tpu as pltpu
from jax.experimental.pallas import tpu_sc as plsc

_K = 256
_R = 128
_N = 32768
_NW = 32
_RPW = _R // _NW
_NCHUNK = _N // 16

_NB1 = 2048
_NB2 = 2048
_NB3 = 1024
_CAP = 1024


def _lane():
    return lax.iota(jnp.int32, 16)


def _monotonic_u32(v):
    u = plsc.bitcast(v, jnp.uint32)
    flip = jnp.where(
        u >= jnp.uint32(0x80000000), jnp.uint32(0xFFFFFFFF), jnp.uint32(0x80000000)
    )
    return u ^ flip


def _vsort_kv(k, v, descending):
    return plsc.sort_key_val(k, v, descending=descending)


def _rev16(x):
    return lax.rev(x, (0,))


def _ce_kv(ak, av, bk, bv, descending):
    m = (ak >= bk) if descending else (ak <= bk)
    return (
        jnp.where(m, ak, bk), jnp.where(m, av, bv),
        jnp.where(m, bk, ak), jnp.where(m, bv, av),
    )


def _sort16(ks, vs, descending):
    ks = list(ks)
    vs = list(vs)
    for i in range(16):
        ks[i], vs[i] = _vsort_kv(ks[i], vs[i], descending)
    size = 2
    while size <= 16:
        half = size // 2
        for base in range(0, 16, size):
            sub_k = ks[base:base + half] + \
                [_rev16(k) for k in ks[base + half:base + size]][::-1]
            sub_v = vs[base:base + half] + \
                [_rev16(v) for v in vs[base + half:base + size]][::-1]
            d = half
            while d >= 1:
                for j in range(size):
                    if (j % (2 * d)) < d:
                        lk, lv, hk, hv = _ce_kv(sub_k[j], sub_v[j],
                                                sub_k[j + d], sub_v[j + d],
                                                descending)
                        sub_k[j], sub_v[j] = lk, lv
                        sub_k[j + d], sub_v[j + d] = hk, hv
                d //= 2
            for j in range(size):
                sub_k[j], sub_v[j] = _vsort_kv(sub_k[j], sub_v[j], descending)
            ks[base:base + size] = sub_k
            vs[base:base + size] = sub_v
        size *= 2
    return ks, vs


def _suffix_select(hist_ref, nb, tots_ref, ts_ref, kp):
    nch = nb // 16
    lane0 = _lane() == 0

    def _tbody(j4, _):
        base = j4 * 4
        vs = [hist_ref[pl.ds((base + u) * 16, 16)] for u in range(4)]
        tots = [jnp.sum(v) for v in vs]
        for u in range(4):
            plsc.store_scatter(
                tots_ref, [jnp.full((16,), base + u, jnp.int32)],
                jnp.broadcast_to(tots[u], (16,)), mask=lane0,
            )
        return 0

    lax.fori_loop(0, nch // 4, _tbody, 0)

    nv = nch // 16
    vs = [tots_ref[pl.ds(jv * 16, 16)] for jv in range(nv)]
    css = [lax.rev(plsc.cumsum(lax.rev(v, (0,))), (0,)) for v in vs]
    tl = [jnp.max(cs) for cs in css]
    run = jnp.int32(0)
    for jv in range(nv - 1, -1, -1):
        sfx = css[jv] + run
        ts_ref[pl.ds(jv * 16, 16)] = sfx
        run = run + tl[jv]

    cnt = jnp.zeros((16,), jnp.int32)
    for jv in range(nch // 16):
        v = ts_ref[pl.ds(jv * 16, 16)]
        cnt = cnt + jnp.where(v >= kp, 1, 0)
    j_sel = jnp.sum(cnt) - 1

    nxt = jnp.minimum(j_sel + 1, nch - 1)
    above_chunks = jnp.max(plsc.load_gather(ts_ref, [jnp.full((16,), nxt, jnp.int32)]))
    above_chunks = jnp.where(j_sel == nch - 1, 0, above_chunks)

    s_chunk = hist_ref[pl.ds(j_sel * 16, 16)]
    sfx_in = lax.rev(plsc.cumsum(lax.rev(s_chunk, (0,))), (0,)) + above_chunks
    b_local = jnp.sum(jnp.where(sfx_in >= kp, 1, 0)) - 1
    bucket = j_sel * 16 + b_local
    above = above_chunks + jnp.sum(jnp.where(_lane() > b_local, s_chunk, 0))
    inside = jnp.sum(jnp.where(_lane() == b_local, s_chunk, 0))
    return bucket, above, inside


def _topk_body(x_hbm, out_hbm, buf, hist1, hist2, hist3, tots, ts,
               sub_key, sub_pos, gt_key, gt_idx, eq_idx,
               cand_key, cand_idx, outrow, dma_sem):
    wid = lax.axis_index("s") * 2 + lax.axis_index("c")
    ones = jnp.ones((16,), jnp.int32)
    zeros16 = jnp.zeros((16,), jnp.int32)
    lane = _lane()

    pltpu.async_copy(x_hbm.at[wid * _RPW], buf.at[pl.ds(0, _N)], dma_sem)

    def row_body(r, _):
        row = wid * _RPW + r
        dbase = (r & 1) * _N

        pltpu.make_async_copy(x_hbm.at[row], buf.at[pl.ds(dbase, _N)],
                              dma_sem).wait()

        @pl.when(r < _RPW - 1)
        def _():
            nbase = ((r + 1) & 1) * _N
            pltpu.async_copy(x_hbm.at[row + 1], buf.at[pl.ds(nbase, _N)],
                             dma_sem)

        def _zbody(i8, _):
            for u in range(8):
                i = i8 * 8 + u
                hist1[pl.ds(i * 16, 16)] = zeros16
                hist2[pl.ds(i * 16, 16)] = zeros16

                @pl.when(i < _NB3 // 16)
                def _():
                    hist3[pl.ds(i * 16, 16)] = zeros16

            return 0

        lax.fori_loop(0, _NB1 // 128, _zbody, 0)

        def _p0(i8, _):
            base = dbase + i8 * 8 * 16
            vs = [buf[pl.ds(base + u * 16, 16)] for u in range(8)]
            mus = [_monotonic_u32(v) for v in vs]
            b1s = [lax.convert_element_type(mu >> 21, jnp.int32) for mu in mus]
            for u in range(8):
                buf[pl.ds(base + u * 16, 16)] = plsc.bitcast(mus[u], jnp.float32)
            for u in range(8):
                plsc.addupdate_scatter(hist1, [b1s[u]], ones)
            return 0

        lax.fori_loop(0, _NCHUNK // 8, _p0, 0)

        bkt1, above1, s1 = _suffix_select(hist1, _NB1, tots, ts, jnp.int32(_K))
        k2 = jnp.int32(_K) - above1
        b1u = lax.convert_element_type(bkt1, jnp.uint32)
        csub = above1 + s1

        def _refine(mu_of, pos_of, nchunks, valid_of):
            ng = (nchunks + 3) >> 2

            def _p2(g, _):
                base = g * 4
                mus = [mu_of(base + u) for u in range(4)]
                ms = [valid_of(base + u) & ((mus[u] >> 21) == b1u)
                      for u in range(4)]
                b2s = [lax.convert_element_type(
                    (mu >> 10) & jnp.uint32(0x7FF), jnp.int32) for mu in mus]
                for u in range(4):
                    plsc.addupdate_scatter(hist2, [b2s[u]], ones, mask=ms[u])
                return 0

            lax.fori_loop(0, ng, _p2, 0)
            bkt2, above2, _ = _suffix_select(hist2, _NB2, tots, ts, k2)
            k3 = k2 - above2
            pre2 = (b1u << 11) | lax.convert_element_type(bkt2, jnp.uint32)

            def _p3(g, _):
                base = g * 4
                mus = [mu_of(base + u) for u in range(4)]
                ms = [valid_of(base + u) & ((mus[u] >> 10) == pre2)
                      for u in range(4)]
                b3s = [lax.convert_element_type(mu & jnp.uint32(0x3FF),
                                                jnp.int32) for mu in mus]
                for u in range(4):
                    plsc.addupdate_scatter(hist3, [b3s[u]], ones, mask=ms[u])
                return 0

            lax.fori_loop(0, ng, _p3, 0)
            bkt3, above3, _ = _suffix_select(hist3, _NB3, tots, ts, k3)
            need = k3 - above3
            c_gt = jnp.int32(_K) - need
            t_u = (pre2 << 10) | lax.convert_element_type(bkt3, jnp.uint32)
            t_splat = jnp.broadcast_to(t_u, (16,))

            def _cbody(g, carry):
                off_gt, off_eq = carry
                base = g * 4
                mus = [mu_of(base + u) for u in range(4)]
                poss = [pos_of(base + u) for u in range(4)]
                m_gts = [valid_of(base + u) & (mus[u] > t_splat)
                         for u in range(4)]
                m_eqs = [valid_of(base + u) & (mus[u] == t_splat)
                         for u in range(4)]
                cs_gts = [plsc.cumsum(ones, mask=m) for m in m_gts]
                cs_eqs = [plsc.cumsum(ones, mask=m) for m in m_eqs]
                n_gts = [plsc.all_reduce_population_count(m) for m in m_gts]
                n_eqs = [plsc.all_reduce_population_count(m) for m in m_eqs]
                for u in range(4):
                    p_gt = off_gt + cs_gts[u] - 1
                    p_eq = off_eq + cs_eqs[u] - 1
                    plsc.store_scatter(gt_key, [p_gt],
                                       plsc.bitcast(mus[u], jnp.int32),
                                       mask=m_gts[u])
                    plsc.store_scatter(gt_idx, [p_gt], poss[u], mask=m_gts[u])
                    plsc.store_scatter(eq_idx, [p_eq], poss[u],
                                       mask=m_eqs[u] & (p_eq < need))
                    off_gt = off_gt + n_gts[u]
                    off_eq = off_eq + n_eqs[u]
                return off_gt, off_eq

            lax.fori_loop(0, ng, _cbody, (zeros16, zeros16))
            return t_splat, need, c_gt

        def _main_path(_):
            def _collect(i8, off):
                base = i8 * 8
                mus = [plsc.bitcast(buf[pl.ds(dbase + (base + u) * 16, 16)],
                                    jnp.uint32) for u in range(8)]
                ms = [(mu >> 21) >= b1u for mu in mus]
                css = [plsc.cumsum(ones, mask=m) for m in ms]
                cnts = [plsc.all_reduce_population_count(m) for m in ms]
                for u in range(8):
                    p = off + css[u] - 1
                    plsc.store_scatter(sub_key, [p],
                                       plsc.bitcast(mus[u], jnp.int32),
                                       mask=ms[u])
                    plsc.store_scatter(sub_pos, [p], (base + u) * 16 + lane,
                                       mask=ms[u])
                    off = off + cnts[u]
                return off

            lax.fori_loop(0, _NCHUNK // 8, _collect, zeros16)

            csub_splat = jnp.broadcast_to(csub, (16,))
            nsub = (csub + 15) >> 4

            def pos_of(i):
                return sub_pos[pl.ds(i * 16, 16)]

            def mu_of(i):
                return plsc.bitcast(sub_key[pl.ds(i * 16, 16)], jnp.uint32)

            def valid_of(i):
                return (i * 16 + lane) < csub_splat

            return _refine(mu_of, pos_of, nsub, valid_of)

        def _fallback(_):
            def mu_of(i):
                return plsc.bitcast(buf[pl.ds(dbase + i * 16, 16)], jnp.uint32)

            def pos_of(i):
                return i * 16 + lane

            def valid_of(i):
                return lane >= 0

            return _refine(mu_of, pos_of, _NCHUNK, valid_of)

        t_splat, need, c_gt = lax.cond(csub <= _CAP, _main_path, _fallback, 0)

        ks = []
        ps = []
        for t in range(_K // 16):
            j = t * 16 + lane
            from_gt = j < c_gt
            jg = jnp.where(from_gt, j, 0)
            je = jnp.maximum(j - c_gt, 0)
            kk = plsc.bitcast(plsc.load_gather(gt_key, [jg]), jnp.uint32)
            key_t = jnp.where(from_gt, kk, t_splat)
            idx_t = jnp.where(
                from_gt,
                plsc.load_gather(gt_idx, [jg]),
                plsc.load_gather(eq_idx, [je]),
            )
            cand_idx[pl.ds(t * 16, 16)] = idx_t
            ks.append(key_t)
            ps.append(j)

        ks, ps = _sort16(ks, ps, descending=True)
        for t in range(_K // 16):
            cand_key[pl.ds(t * 16, 16)] = plsc.bitcast(ks[t], jnp.int32)

        sts = []
        for t in range(_K // 16):
            qidx = t * 16 + lane
            sh = plsc.bitcast(
                plsc.load_gather(cand_key, [jnp.maximum(qidx - 1, 0)]),
                jnp.uint32,
            )
            d = ks[t] != sh
            if t == 0:
                d = d | (lane == 0)
            sts.append(jnp.where(d, qidx, 0))
        comp = []
        carry = jnp.int32(0)
        for t in range(_K // 16):
            rs = jnp.maximum(plsc.cummax(sts[t]), carry)
            comp.append((rs << 8) + ps[t])
            carry = jnp.maximum(carry, jnp.max(sts[t]))
        _, ps2 = _sort16(comp, ps, descending=False)

        for t in range(_K // 16):
            outrow[pl.ds(t * 16, 16)] = plsc.load_gather(cand_idx, [ps2[t]])

        pltpu.sync_copy(outrow, out_hbm.at[row])
        return 0

    lax.fori_loop(0, _RPW, row_body, 0)


@jax.jit
def kernel(x):
    mesh = plsc.VectorSubcoreMesh(core_axis_name="c", subcore_axis_name="s")
    f = functools.partial(
        pl.kernel,
        mesh=mesh,
        compiler_params=pltpu.CompilerParams(needs_layout_passes=False),
        out_type=jax.ShapeDtypeStruct((_R, _K), jnp.int32),
        scratch_types=[
            pltpu.VMEM((2 * _N,), jnp.float32),
            pltpu.VMEM((_NB1,), jnp.int32),
            pltpu.VMEM((_NB2,), jnp.int32),
            pltpu.VMEM((_NB3,), jnp.int32),
            pltpu.VMEM((256,), jnp.int32),
            pltpu.VMEM((256,), jnp.int32),
            pltpu.VMEM((_CAP + 16,), jnp.int32),
            pltpu.VMEM((_CAP + 16,), jnp.int32),
            pltpu.VMEM((512,), jnp.int32),
            pltpu.VMEM((512,), jnp.int32),
            pltpu.VMEM((512,), jnp.int32),
            pltpu.VMEM((_K,), jnp.int32),
            pltpu.VMEM((_K,), jnp.int32),
            pltpu.VMEM((_K,), jnp.int32),
            pltpu.SemaphoreType.DMA,
        ],
    )(_topk_body)
    return f(x)


# speedup vs baseline: 1.2203x; 1.1109x over previous
import functools

import jax
import jax.numpy as jnp
from jax import lax
from jax.experimental import pallas as pl
from jax.experimental.pallas import tpu as pltpu
from jax.experimental.pallas import tpu_sc as plsc

_K = 256
_R = 128
_N = 32768
_NW = 32
_RPW = _R // _NW
_NCHUNK = _N // 16

_NB1 = 2048
_NB2 = 2048
_NB3 = 1024
_CAP = 1024


def _lane():
    return lax.iota(jnp.int32, 16)


def _monotonic_u32(v):
    u = plsc.bitcast(v, jnp.uint32)
    flip = jnp.where(
        u >= jnp.uint32(0x80000000), jnp.uint32(0xFFFFFFFF), jnp.uint32(0x80000000)
    )
    return u ^ flip


def _vsort_kv(k, v, descending):
    return plsc.sort_key_val(k, v, descending=descending)


def _rev16(x):
    return lax.rev(x, (0,))


def _ce_kv(ak, av, bk, bv, descending):
    m = (ak >= bk) if descending else (ak <= bk)
    return (
        jnp.where(m, ak, bk), jnp.where(m, av, bv),
        jnp.where(m, bk, ak), jnp.where(m, bv, av),
    )


def _sort16(ks, vs, descending):
    ks = list(ks)
    vs = list(vs)
    for i in range(16):
        ks[i], vs[i] = _vsort_kv(ks[i], vs[i], descending)
    size = 2
    while size <= 16:
        half = size // 2
        for base in range(0, 16, size):
            sub_k = ks[base:base + half] + \
                [_rev16(k) for k in ks[base + half:base + size]][::-1]
            sub_v = vs[base:base + half] + \
                [_rev16(v) for v in vs[base + half:base + size]][::-1]
            d = half
            while d >= 1:
                for j in range(size):
                    if (j % (2 * d)) < d:
                        lk, lv, hk, hv = _ce_kv(sub_k[j], sub_v[j],
                                                sub_k[j + d], sub_v[j + d],
                                                descending)
                        sub_k[j], sub_v[j] = lk, lv
                        sub_k[j + d], sub_v[j + d] = hk, hv
                d //= 2
            for j in range(size):
                sub_k[j], sub_v[j] = _vsort_kv(sub_k[j], sub_v[j], descending)
            ks[base:base + size] = sub_k
            vs[base:base + size] = sub_v
        size *= 2
    return ks, vs


def _suffix_select(hist_ref, nb, tots_ref, ts_ref, kp):
    nch = nb // 16
    lane0 = _lane() == 0

    def _tbody(j4, _):
        base = j4 * 4
        vs = [hist_ref[pl.ds((base + u) * 16, 16)] for u in range(4)]
        tots = [jnp.sum(v) for v in vs]
        for u in range(4):
            plsc.store_scatter(
                tots_ref, [jnp.full((16,), base + u, jnp.int32)],
                jnp.broadcast_to(tots[u], (16,)), mask=lane0,
            )
        return 0

    lax.fori_loop(0, nch // 4, _tbody, 0)

    nv = nch // 16
    vs = [tots_ref[pl.ds(jv * 16, 16)] for jv in range(nv)]
    css = [lax.rev(plsc.cumsum(lax.rev(v, (0,))), (0,)) for v in vs]
    tl = [jnp.max(cs) for cs in css]
    run = jnp.int32(0)
    for jv in range(nv - 1, -1, -1):
        sfx = css[jv] + run
        ts_ref[pl.ds(jv * 16, 16)] = sfx
        run = run + tl[jv]

    cnt = jnp.zeros((16,), jnp.int32)
    for jv in range(nch // 16):
        v = ts_ref[pl.ds(jv * 16, 16)]
        cnt = cnt + jnp.where(v >= kp, 1, 0)
    j_sel = jnp.sum(cnt) - 1

    nxt = jnp.minimum(j_sel + 1, nch - 1)
    above_chunks = jnp.max(plsc.load_gather(ts_ref, [jnp.full((16,), nxt, jnp.int32)]))
    above_chunks = jnp.where(j_sel == nch - 1, 0, above_chunks)

    s_chunk = hist_ref[pl.ds(j_sel * 16, 16)]
    sfx_in = lax.rev(plsc.cumsum(lax.rev(s_chunk, (0,))), (0,)) + above_chunks
    b_local = jnp.sum(jnp.where(sfx_in >= kp, 1, 0)) - 1
    bucket = j_sel * 16 + b_local
    above = above_chunks + jnp.sum(jnp.where(_lane() > b_local, s_chunk, 0))
    inside = jnp.sum(jnp.where(_lane() == b_local, s_chunk, 0))
    return bucket, above, inside


def _topk_body(x_hbm, out_hbm, buf, hist1, hist2, hist3, tots, ts,
               sub_key, sub_pos, gt_key, gt_idx, eq_idx,
               cand_key, cand_idx, outrow, dma_sem):
    wid = lax.axis_index("s") * 2 + lax.axis_index("c")
    ones = jnp.ones((16,), jnp.int32)
    zeros16 = jnp.zeros((16,), jnp.int32)
    lane = _lane()

    pltpu.async_copy(x_hbm.at[wid * _RPW], buf.at[pl.ds(0, _N)], dma_sem)

    def row_body(r, _):
        row = wid * _RPW + r
        dbase = (r & 1) * _N

        def _zbody(i8, _):
            for u in range(8):
                i = i8 * 8 + u
                hist1[pl.ds(i * 16, 16)] = zeros16
                hist2[pl.ds(i * 16, 16)] = zeros16

                @pl.when(i < _NB3 // 16)
                def _():
                    hist3[pl.ds(i * 16, 16)] = zeros16

            return 0

        lax.fori_loop(0, _NB1 // 128, _zbody, 0)

        pltpu.make_async_copy(x_hbm.at[row], buf.at[pl.ds(dbase, _N)],
                              dma_sem).wait()

        @pl.when(r < _RPW - 1)
        def _():
            nbase = ((r + 1) & 1) * _N
            pltpu.async_copy(x_hbm.at[row + 1], buf.at[pl.ds(nbase, _N)],
                             dma_sem)


        def _p0(i16, _):
            base = dbase + i16 * 16 * 16
            vs = [buf[pl.ds(base + u * 16, 16)] for u in range(16)]
            mus = [_monotonic_u32(v) for v in vs]
            b1s = [lax.convert_element_type(mu >> 21, jnp.int32) for mu in mus]
            for u in range(16):
                buf[pl.ds(base + u * 16, 16)] = plsc.bitcast(mus[u], jnp.float32)
            for u in range(16):
                plsc.addupdate_scatter(hist1, [b1s[u]], ones)
            return 0

        lax.fori_loop(0, _NCHUNK // 16, _p0, 0)

        bkt1, above1, s1 = _suffix_select(hist1, _NB1, tots, ts, jnp.int32(_K))
        k2 = jnp.int32(_K) - above1
        b1u = lax.convert_element_type(bkt1, jnp.uint32)
        csub = above1 + s1

        def _refine(mu_of, pos_of, nchunks, valid_of):
            ng = (nchunks + 3) >> 2

            def _p2(g, _):
                base = g * 4
                mus = [mu_of(base + u) for u in range(4)]
                ms = [valid_of(base + u) & ((mus[u] >> 21) == b1u)
                      for u in range(4)]
                b2s = [lax.convert_element_type(
                    (mu >> 10) & jnp.uint32(0x7FF), jnp.int32) for mu in mus]
                for u in range(4):
                    plsc.addupdate_scatter(hist2, [b2s[u]], ones, mask=ms[u])
                return 0

            lax.fori_loop(0, ng, _p2, 0)
            bkt2, above2, _ = _suffix_select(hist2, _NB2, tots, ts, k2)
            k3 = k2 - above2
            pre2 = (b1u << 11) | lax.convert_element_type(bkt2, jnp.uint32)

            def _p3(g, _):
                base = g * 4
                mus = [mu_of(base + u) for u in range(4)]
                ms = [valid_of(base + u) & ((mus[u] >> 10) == pre2)
                      for u in range(4)]
                b3s = [lax.convert_element_type(mu & jnp.uint32(0x3FF),
                                                jnp.int32) for mu in mus]
                for u in range(4):
                    plsc.addupdate_scatter(hist3, [b3s[u]], ones, mask=ms[u])
                return 0

            lax.fori_loop(0, ng, _p3, 0)
            bkt3, above3, _ = _suffix_select(hist3, _NB3, tots, ts, k3)
            need = k3 - above3
            c_gt = jnp.int32(_K) - need
            t_u = (pre2 << 10) | lax.convert_element_type(bkt3, jnp.uint32)
            t_splat = jnp.broadcast_to(t_u, (16,))

            def _cbody(g, carry):
                off_gt, off_eq = carry
                base = g * 4
                mus = [mu_of(base + u) for u in range(4)]
                poss = [pos_of(base + u) for u in range(4)]
                m_gts = [valid_of(base + u) & (mus[u] > t_splat)
                         for u in range(4)]
                m_eqs = [valid_of(base + u) & (mus[u] == t_splat)
                         for u in range(4)]
                cs_gts = [plsc.cumsum(ones, mask=m) for m in m_gts]
                cs_eqs = [plsc.cumsum(ones, mask=m) for m in m_eqs]
                n_gts = [plsc.all_reduce_population_count(m) for m in m_gts]
                n_eqs = [plsc.all_reduce_population_count(m) for m in m_eqs]
                for u in range(4):
                    p_gt = off_gt + cs_gts[u] - 1
                    p_eq = off_eq + cs_eqs[u] - 1
                    plsc.store_scatter(gt_key, [p_gt],
                                       plsc.bitcast(mus[u], jnp.int32),
                                       mask=m_gts[u])
                    plsc.store_scatter(gt_idx, [p_gt], poss[u], mask=m_gts[u])
                    plsc.store_scatter(eq_idx, [p_eq], poss[u],
                                       mask=m_eqs[u] & (p_eq < need))
                    off_gt = off_gt + n_gts[u]
                    off_eq = off_eq + n_eqs[u]
                return off_gt, off_eq

            lax.fori_loop(0, ng, _cbody, (zeros16, zeros16))
            return t_splat, need, c_gt

        def _main_path(_):
            def _collect(i16, off):
                base = i16 * 16
                mus = [plsc.bitcast(buf[pl.ds(dbase + (base + u) * 16, 16)],
                                    jnp.uint32) for u in range(16)]
                ms = [(mu >> 21) >= b1u for mu in mus]
                css = [plsc.cumsum(ones, mask=m) for m in ms]
                cnts = [plsc.all_reduce_population_count(m) for m in ms]
                for u in range(16):
                    p = off + css[u] - 1
                    plsc.store_scatter(sub_key, [p],
                                       plsc.bitcast(mus[u], jnp.int32),
                                       mask=ms[u])
                    plsc.store_scatter(sub_pos, [p], (base + u) * 16 + lane,
                                       mask=ms[u])
                    off = off + cnts[u]
                return off

            lax.fori_loop(0, _NCHUNK // 16, _collect, zeros16)

            csub_splat = jnp.broadcast_to(csub, (16,))
            nsub = (csub + 15) >> 4

            def pos_of(i):
                return sub_pos[pl.ds(i * 16, 16)]

            def mu_of(i):
                return plsc.bitcast(sub_key[pl.ds(i * 16, 16)], jnp.uint32)

            def valid_of(i):
                return (i * 16 + lane) < csub_splat

            return _refine(mu_of, pos_of, nsub, valid_of)

        def _fallback(_):
            def mu_of(i):
                return plsc.bitcast(buf[pl.ds(dbase + i * 16, 16)], jnp.uint32)

            def pos_of(i):
                return i * 16 + lane

            def valid_of(i):
                return lane >= 0

            return _refine(mu_of, pos_of, _NCHUNK, valid_of)

        t_splat, need, c_gt = lax.cond(csub <= _CAP, _main_path, _fallback, 0)

        ks = []
        ps = []
        for t in range(_K // 16):
            j = t * 16 + lane
            from_gt = j < c_gt
            jg = jnp.where(from_gt, j, 0)
            je = jnp.maximum(j - c_gt, 0)
            kk = plsc.bitcast(plsc.load_gather(gt_key, [jg]), jnp.uint32)
            key_t = jnp.where(from_gt, kk, t_splat)
            idx_t = jnp.where(
                from_gt,
                plsc.load_gather(gt_idx, [jg]),
                plsc.load_gather(eq_idx, [je]),
            )
            cand_idx[pl.ds(t * 16, 16)] = idx_t
            ks.append(key_t)
            ps.append(j)

        ks, ps = _sort16(ks, ps, descending=True)
        for t in range(_K // 16):
            cand_key[pl.ds(t * 16, 16)] = plsc.bitcast(ks[t], jnp.int32)

        sts = []
        for t in range(_K // 16):
            qidx = t * 16 + lane
            sh = plsc.bitcast(
                plsc.load_gather(cand_key, [jnp.maximum(qidx - 1, 0)]),
                jnp.uint32,
            )
            d = ks[t] != sh
            if t == 0:
                d = d | (lane == 0)
            sts.append(jnp.where(d, qidx, 0))
        comp = []
        carry = jnp.int32(0)
        for t in range(_K // 16):
            rs = jnp.maximum(plsc.cummax(sts[t]), carry)
            comp.append((rs << 8) + ps[t])
            carry = jnp.maximum(carry, jnp.max(sts[t]))
        _, ps2 = _sort16(comp, ps, descending=False)

        for t in range(_K // 16):
            outrow[pl.ds(t * 16, 16)] = plsc.load_gather(cand_idx, [ps2[t]])

        pltpu.sync_copy(outrow, out_hbm.at[row])
        return 0

    lax.fori_loop(0, _RPW, row_body, 0)


@jax.jit
def kernel(x):
    mesh = plsc.VectorSubcoreMesh(core_axis_name="c", subcore_axis_name="s")
    f = functools.partial(
        pl.kernel,
        mesh=mesh,
        compiler_params=pltpu.CompilerParams(needs_layout_passes=False),
        out_type=jax.ShapeDtypeStruct((_R, _K), jnp.int32),
        scratch_types=[
            pltpu.VMEM((2 * _N,), jnp.float32),
            pltpu.VMEM((_NB1,), jnp.int32),
            pltpu.VMEM((_NB2,), jnp.int32),
            pltpu.VMEM((_NB3,), jnp.int32),
            pltpu.VMEM((256,), jnp.int32),
            pltpu.VMEM((256,), jnp.int32),
            pltpu.VMEM((_CAP + 16,), jnp.int32),
            pltpu.VMEM((_CAP + 16,), jnp.int32),
            pltpu.VMEM((512,), jnp.int32),
            pltpu.VMEM((512,), jnp.int32),
            pltpu.VMEM((512,), jnp.int32),
            pltpu.VMEM((_K,), jnp.int32),
            pltpu.VMEM((_K,), jnp.int32),
            pltpu.VMEM((_K,), jnp.int32),
            pltpu.SemaphoreType.DMA,
        ],
    )(_topk_body)
    return f(x)


# speedup vs baseline: 1.2703x; 1.0410x over previous
import functools

import jax
import jax.numpy as jnp
from jax import lax
from jax.experimental import pallas as pl
from jax.experimental.pallas import tpu as pltpu
from jax.experimental.pallas import tpu_sc as plsc

_K = 256
_R = 128
_N = 32768
_NW = 32
_RPW = _R // _NW
_NCHUNK = _N // 16

_NB1 = 2048
_NB2 = 2048
_NB3 = 1024
_CAP = 1024


def _lane():
    return lax.iota(jnp.int32, 16)


def _monotonic_u32(v):
    u = plsc.bitcast(v, jnp.uint32)
    flip = jnp.where(
        u >= jnp.uint32(0x80000000), jnp.uint32(0xFFFFFFFF), jnp.uint32(0x80000000)
    )
    return u ^ flip


def _vsort_kv(k, v, descending):
    return plsc.sort_key_val(k, v, descending=descending)


def _rev16(x):
    return lax.rev(x, (0,))


def _ce_kv(ak, av, bk, bv, descending):
    m = (ak >= bk) if descending else (ak <= bk)
    return (
        jnp.where(m, ak, bk), jnp.where(m, av, bv),
        jnp.where(m, bk, ak), jnp.where(m, bv, av),
    )


def _sort16(ks, vs, descending):
    ks = list(ks)
    vs = list(vs)
    for i in range(16):
        ks[i], vs[i] = _vsort_kv(ks[i], vs[i], descending)
    size = 2
    while size <= 16:
        half = size // 2
        for base in range(0, 16, size):
            sub_k = ks[base:base + half] + \
                [_rev16(k) for k in ks[base + half:base + size]][::-1]
            sub_v = vs[base:base + half] + \
                [_rev16(v) for v in vs[base + half:base + size]][::-1]
            d = half
            while d >= 1:
                for j in range(size):
                    if (j % (2 * d)) < d:
                        lk, lv, hk, hv = _ce_kv(sub_k[j], sub_v[j],
                                                sub_k[j + d], sub_v[j + d],
                                                descending)
                        sub_k[j], sub_v[j] = lk, lv
                        sub_k[j + d], sub_v[j + d] = hk, hv
                d //= 2
            for j in range(size):
                sub_k[j], sub_v[j] = _vsort_kv(sub_k[j], sub_v[j], descending)
            ks[base:base + size] = sub_k
            vs[base:base + size] = sub_v
        size *= 2
    return ks, vs


def _suffix_select(hist_ref, nb, tots_ref, ts_ref, kp):
    nch = nb // 16
    lane0 = _lane() == 0

    def _tbody(j4, _):
        base = j4 * 4
        vs = [hist_ref[pl.ds((base + u) * 16, 16)] for u in range(4)]
        tots = [jnp.sum(v) for v in vs]
        for u in range(4):
            plsc.store_scatter(
                tots_ref, [jnp.full((16,), base + u, jnp.int32)],
                jnp.broadcast_to(tots[u], (16,)), mask=lane0,
            )
        return 0

    lax.fori_loop(0, nch // 4, _tbody, 0)

    nv = nch // 16
    vs = [tots_ref[pl.ds(jv * 16, 16)] for jv in range(nv)]
    css = [lax.rev(plsc.cumsum(lax.rev(v, (0,))), (0,)) for v in vs]
    tl = [jnp.max(cs) for cs in css]
    run = jnp.int32(0)
    for jv in range(nv - 1, -1, -1):
        sfx = css[jv] + run
        ts_ref[pl.ds(jv * 16, 16)] = sfx
        run = run + tl[jv]

    cnt = jnp.zeros((16,), jnp.int32)
    for jv in range(nch // 16):
        v = ts_ref[pl.ds(jv * 16, 16)]
        cnt = cnt + jnp.where(v >= kp, 1, 0)
    j_sel = jnp.sum(cnt) - 1

    nxt = jnp.minimum(j_sel + 1, nch - 1)
    above_chunks = jnp.max(plsc.load_gather(ts_ref, [jnp.full((16,), nxt, jnp.int32)]))
    above_chunks = jnp.where(j_sel == nch - 1, 0, above_chunks)

    s_chunk = hist_ref[pl.ds(j_sel * 16, 16)]
    sfx_in = lax.rev(plsc.cumsum(lax.rev(s_chunk, (0,))), (0,)) + above_chunks
    b_local = jnp.sum(jnp.where(sfx_in >= kp, 1, 0)) - 1
    bucket = j_sel * 16 + b_local
    above = above_chunks + jnp.sum(jnp.where(_lane() > b_local, s_chunk, 0))
    inside = jnp.sum(jnp.where(_lane() == b_local, s_chunk, 0))
    return bucket, above, inside


def _topk_body(x_hbm, out_hbm, buf, hist1, hist2, hist3, tots, ts,
               sub_key, sub_pos, gt_key, gt_idx, eq_idx,
               cand_key, cand_idx, outrow, dma_sem):
    wid = lax.axis_index("s") * 2 + lax.axis_index("c")
    ones = jnp.ones((16,), jnp.int32)
    zeros16 = jnp.zeros((16,), jnp.int32)
    lane = _lane()

    pltpu.async_copy(x_hbm.at[wid * _RPW], buf.at[pl.ds(0, _N)], dma_sem)

    def row_body(r, _):
        row = wid * _RPW + r
        dbase = (r & 1) * _N

        def _zbody(i8, _):
            for u in range(8):
                i = i8 * 8 + u
                hist1[pl.ds(i * 16, 16)] = zeros16
                hist2[pl.ds(i * 16, 16)] = zeros16

                @pl.when(i < _NB3 // 16)
                def _():
                    hist3[pl.ds(i * 16, 16)] = zeros16

            return 0

        lax.fori_loop(0, _NB1 // 128, _zbody, 0)

        pltpu.make_async_copy(x_hbm.at[row], buf.at[pl.ds(dbase, _N)],
                              dma_sem).wait()

        @pl.when(r < _RPW - 1)
        def _():
            nbase = ((r + 1) & 1) * _N
            pltpu.async_copy(x_hbm.at[row + 1], buf.at[pl.ds(nbase, _N)],
                             dma_sem)


        def _p0(i32, _):
            base = dbase + i32 * 32 * 16
            vs = [buf[pl.ds(base + u * 16, 16)] for u in range(32)]
            mus = [_monotonic_u32(v) for v in vs]
            b1s = [lax.convert_element_type(mu >> 21, jnp.int32) for mu in mus]
            for u in range(32):
                buf[pl.ds(base + u * 16, 16)] = plsc.bitcast(mus[u], jnp.float32)
            for u in range(32):
                plsc.addupdate_scatter(hist1, [b1s[u]], ones)
            return 0

        lax.fori_loop(0, _NCHUNK // 32, _p0, 0)

        bkt1, above1, s1 = _suffix_select(hist1, _NB1, tots, ts, jnp.int32(_K))
        k2 = jnp.int32(_K) - above1
        b1u = lax.convert_element_type(bkt1, jnp.uint32)
        csub = above1 + s1

        def _refine(mu_of, pos_of, nchunks, valid_of):
            ng = (nchunks + 3) >> 2

            def _p2(g, _):
                base = g * 4
                mus = [mu_of(base + u) for u in range(4)]
                ms = [valid_of(base + u) & ((mus[u] >> 21) == b1u)
                      for u in range(4)]
                b2s = [lax.convert_element_type(
                    (mu >> 10) & jnp.uint32(0x7FF), jnp.int32) for mu in mus]
                for u in range(4):
                    plsc.addupdate_scatter(hist2, [b2s[u]], ones, mask=ms[u])
                return 0

            lax.fori_loop(0, ng, _p2, 0)
            bkt2, above2, _ = _suffix_select(hist2, _NB2, tots, ts, k2)
            k3 = k2 - above2
            pre2 = (b1u << 11) | lax.convert_element_type(bkt2, jnp.uint32)

            def _p3(g, _):
                base = g * 4
                mus = [mu_of(base + u) for u in range(4)]
                ms = [valid_of(base + u) & ((mus[u] >> 10) == pre2)
                      for u in range(4)]
                b3s = [lax.convert_element_type(mu & jnp.uint32(0x3FF),
                                                jnp.int32) for mu in mus]
                for u in range(4):
                    plsc.addupdate_scatter(hist3, [b3s[u]], ones, mask=ms[u])
                return 0

            lax.fori_loop(0, ng, _p3, 0)
            bkt3, above3, _ = _suffix_select(hist3, _NB3, tots, ts, k3)
            need = k3 - above3
            c_gt = jnp.int32(_K) - need
            t_u = (pre2 << 10) | lax.convert_element_type(bkt3, jnp.uint32)
            t_splat = jnp.broadcast_to(t_u, (16,))

            def _cbody(g, carry):
                off_gt, off_eq = carry
                base = g * 4
                mus = [mu_of(base + u) for u in range(4)]
                poss = [pos_of(base + u) for u in range(4)]
                m_gts = [valid_of(base + u) & (mus[u] > t_splat)
                         for u in range(4)]
                m_eqs = [valid_of(base + u) & (mus[u] == t_splat)
                         for u in range(4)]
                cs_gts = [plsc.cumsum(ones, mask=m) for m in m_gts]
                cs_eqs = [plsc.cumsum(ones, mask=m) for m in m_eqs]
                n_gts = [plsc.all_reduce_population_count(m) for m in m_gts]
                n_eqs = [plsc.all_reduce_population_count(m) for m in m_eqs]
                for u in range(4):
                    p_gt = off_gt + cs_gts[u] - 1
                    p_eq = off_eq + cs_eqs[u] - 1
                    plsc.store_scatter(gt_key, [p_gt],
                                       plsc.bitcast(mus[u], jnp.int32),
                                       mask=m_gts[u])
                    plsc.store_scatter(gt_idx, [p_gt], poss[u], mask=m_gts[u])
                    plsc.store_scatter(eq_idx, [p_eq], poss[u],
                                       mask=m_eqs[u] & (p_eq < need))
                    off_gt = off_gt + n_gts[u]
                    off_eq = off_eq + n_eqs[u]
                return off_gt, off_eq

            lax.fori_loop(0, ng, _cbody, (zeros16, zeros16))
            return t_splat, need, c_gt

        def _main_path(_):
            def _collect(i32, off):
                base = i32 * 32
                mus = [plsc.bitcast(buf[pl.ds(dbase + (base + u) * 16, 16)],
                                    jnp.uint32) for u in range(32)]
                ms = [(mu >> 21) >= b1u for mu in mus]
                css = [plsc.cumsum(ones, mask=m) for m in ms]
                cnts = [plsc.all_reduce_population_count(m) for m in ms]
                for u in range(32):
                    p = off + css[u] - 1
                    plsc.store_scatter(sub_key, [p],
                                       plsc.bitcast(mus[u], jnp.int32),
                                       mask=ms[u])
                    plsc.store_scatter(sub_pos, [p], (base + u) * 16 + lane,
                                       mask=ms[u])
                    off = off + cnts[u]
                return off

            lax.fori_loop(0, _NCHUNK // 32, _collect, zeros16)

            csub_splat = jnp.broadcast_to(csub, (16,))
            nsub = (csub + 15) >> 4

            def pos_of(i):
                return sub_pos[pl.ds(i * 16, 16)]

            def mu_of(i):
                return plsc.bitcast(sub_key[pl.ds(i * 16, 16)], jnp.uint32)

            def valid_of(i):
                return (i * 16 + lane) < csub_splat

            return _refine(mu_of, pos_of, nsub, valid_of)

        def _fallback(_):
            def mu_of(i):
                return plsc.bitcast(buf[pl.ds(dbase + i * 16, 16)], jnp.uint32)

            def pos_of(i):
                return i * 16 + lane

            def valid_of(i):
                return lane >= 0

            return _refine(mu_of, pos_of, _NCHUNK, valid_of)

        t_splat, need, c_gt = lax.cond(csub <= _CAP, _main_path, _fallback, 0)

        ks = []
        ps = []
        for t in range(_K // 16):
            j = t * 16 + lane
            from_gt = j < c_gt
            jg = jnp.where(from_gt, j, 0)
            je = jnp.maximum(j - c_gt, 0)
            kk = plsc.bitcast(plsc.load_gather(gt_key, [jg]), jnp.uint32)
            key_t = jnp.where(from_gt, kk, t_splat)
            idx_t = jnp.where(
                from_gt,
                plsc.load_gather(gt_idx, [jg]),
                plsc.load_gather(eq_idx, [je]),
            )
            cand_idx[pl.ds(t * 16, 16)] = idx_t
            ks.append(key_t)
            ps.append(j)

        ks, ps = _sort16(ks, ps, descending=True)
        for t in range(_K // 16):
            cand_key[pl.ds(t * 16, 16)] = plsc.bitcast(ks[t], jnp.int32)

        sts = []
        for t in range(_K // 16):
            qidx = t * 16 + lane
            sh = plsc.bitcast(
                plsc.load_gather(cand_key, [jnp.maximum(qidx - 1, 0)]),
                jnp.uint32,
            )
            d = ks[t] != sh
            if t == 0:
                d = d | (lane == 0)
            sts.append(jnp.where(d, qidx, 0))
        comp = []
        carry = jnp.int32(0)
        for t in range(_K // 16):
            rs = jnp.maximum(plsc.cummax(sts[t]), carry)
            comp.append((rs << 8) + ps[t])
            carry = jnp.maximum(carry, jnp.max(sts[t]))
        _, ps2 = _sort16(comp, ps, descending=False)

        for t in range(_K // 16):
            outrow[pl.ds(t * 16, 16)] = plsc.load_gather(cand_idx, [ps2[t]])

        pltpu.sync_copy(outrow, out_hbm.at[row])
        return 0

    lax.fori_loop(0, _RPW, row_body, 0)


@jax.jit
def kernel(x):
    mesh = plsc.VectorSubcoreMesh(core_axis_name="c", subcore_axis_name="s")
    f = functools.partial(
        pl.kernel,
        mesh=mesh,
        compiler_params=pltpu.CompilerParams(needs_layout_passes=False),
        out_type=jax.ShapeDtypeStruct((_R, _K), jnp.int32),
        scratch_types=[
            pltpu.VMEM((2 * _N,), jnp.float32),
            pltpu.VMEM((_NB1,), jnp.int32),
            pltpu.VMEM((_NB2,), jnp.int32),
            pltpu.VMEM((_NB3,), jnp.int32),
            pltpu.VMEM((256,), jnp.int32),
            pltpu.VMEM((256,), jnp.int32),
            pltpu.VMEM((_CAP + 16,), jnp.int32),
            pltpu.VMEM((_CAP + 16,), jnp.int32),
            pltpu.VMEM((512,), jnp.int32),
            pltpu.VMEM((512,), jnp.int32),
            pltpu.VMEM((512,), jnp.int32),
            pltpu.VMEM((_K,), jnp.int32),
            pltpu.VMEM((_K,), jnp.int32),
            pltpu.VMEM((_K,), jnp.int32),
            pltpu.SemaphoreType.DMA,
        ],
    )(_topk_body)
    return f(x)


# speedup vs baseline: 1.2727x; 1.0019x over previous
import functools

import jax
import jax.numpy as jnp
from jax import lax
from jax.experimental import pallas as pl
from jax.experimental.pallas import tpu as pltpu
from jax.experimental.pallas import tpu_sc as plsc

_K = 256
_R = 128
_N = 32768
_NW = 32
_RPW = _R // _NW
_NCHUNK = _N // 16

_NB1 = 2048
_NB2 = 2048
_NB3 = 1024
_CAP = 1024


def _lane():
    return lax.iota(jnp.int32, 16)


def _monotonic_u32(v):
    u = plsc.bitcast(v, jnp.uint32)
    flip = jnp.where(
        u >= jnp.uint32(0x80000000), jnp.uint32(0xFFFFFFFF), jnp.uint32(0x80000000)
    )
    return u ^ flip


def _vsort_kv(k, v, descending):
    return plsc.sort_key_val(k, v, descending=descending)


def _rev16(x):
    return lax.rev(x, (0,))


def _ce_kv(ak, av, bk, bv, descending):
    m = (ak >= bk) if descending else (ak <= bk)
    return (
        jnp.where(m, ak, bk), jnp.where(m, av, bv),
        jnp.where(m, bk, ak), jnp.where(m, bv, av),
    )


def _sort16(ks, vs, descending):
    ks = list(ks)
    vs = list(vs)
    for i in range(16):
        ks[i], vs[i] = _vsort_kv(ks[i], vs[i], descending)
    size = 2
    while size <= 16:
        half = size // 2
        for base in range(0, 16, size):
            sub_k = ks[base:base + half] + \
                [_rev16(k) for k in ks[base + half:base + size]][::-1]
            sub_v = vs[base:base + half] + \
                [_rev16(v) for v in vs[base + half:base + size]][::-1]
            d = half
            while d >= 1:
                for j in range(size):
                    if (j % (2 * d)) < d:
                        lk, lv, hk, hv = _ce_kv(sub_k[j], sub_v[j],
                                                sub_k[j + d], sub_v[j + d],
                                                descending)
                        sub_k[j], sub_v[j] = lk, lv
                        sub_k[j + d], sub_v[j + d] = hk, hv
                d //= 2
            for j in range(size):
                sub_k[j], sub_v[j] = _vsort_kv(sub_k[j], sub_v[j], descending)
            ks[base:base + size] = sub_k
            vs[base:base + size] = sub_v
        size *= 2
    return ks, vs


def _suffix_select(hist_ref, nb, tots_ref, ts_ref, kp):
    nch = nb // 16
    lane0 = _lane() == 0

    def _tbody(j4, _):
        base = j4 * 4
        vs = [hist_ref[pl.ds((base + u) * 16, 16)] for u in range(4)]
        tots = [jnp.sum(v) for v in vs]
        for u in range(4):
            plsc.store_scatter(
                tots_ref, [jnp.full((16,), base + u, jnp.int32)],
                jnp.broadcast_to(tots[u], (16,)), mask=lane0,
            )
        return 0

    lax.fori_loop(0, nch // 4, _tbody, 0)

    nv = nch // 16
    vs = [tots_ref[pl.ds(jv * 16, 16)] for jv in range(nv)]
    css = [lax.rev(plsc.cumsum(lax.rev(v, (0,))), (0,)) for v in vs]
    tl = [jnp.max(cs) for cs in css]
    run = jnp.int32(0)
    for jv in range(nv - 1, -1, -1):
        sfx = css[jv] + run
        ts_ref[pl.ds(jv * 16, 16)] = sfx
        run = run + tl[jv]

    cnt = jnp.zeros((16,), jnp.int32)
    for jv in range(nch // 16):
        v = ts_ref[pl.ds(jv * 16, 16)]
        cnt = cnt + jnp.where(v >= kp, 1, 0)
    j_sel = jnp.sum(cnt) - 1

    nxt = jnp.minimum(j_sel + 1, nch - 1)
    above_chunks = jnp.max(plsc.load_gather(ts_ref, [jnp.full((16,), nxt, jnp.int32)]))
    above_chunks = jnp.where(j_sel == nch - 1, 0, above_chunks)

    s_chunk = hist_ref[pl.ds(j_sel * 16, 16)]
    sfx_in = lax.rev(plsc.cumsum(lax.rev(s_chunk, (0,))), (0,)) + above_chunks
    b_local = jnp.sum(jnp.where(sfx_in >= kp, 1, 0)) - 1
    bucket = j_sel * 16 + b_local
    above = above_chunks + jnp.sum(jnp.where(_lane() > b_local, s_chunk, 0))
    inside = jnp.sum(jnp.where(_lane() == b_local, s_chunk, 0))
    return bucket, above, inside


def _topk_body(x_hbm, out_hbm, buf, hist1, hist2, hist3, tots, ts,
               sub_key, sub_pos, gt_key, gt_idx, eq_idx,
               cand_key, cand_idx, outrow, dma_sem):
    wid = lax.axis_index("s") * 2 + lax.axis_index("c")
    ones = jnp.ones((16,), jnp.int32)
    zeros16 = jnp.zeros((16,), jnp.int32)
    lane = _lane()

    pltpu.async_copy(x_hbm.at[wid * _RPW], buf.at[pl.ds(0, _N)], dma_sem)

    def row_body(r, _):
        row = wid * _RPW + r
        dbase = (r & 1) * _N

        def _zbody(i8, _):
            for u in range(8):
                i = i8 * 8 + u
                hist1[pl.ds(i * 16, 16)] = zeros16
                hist2[pl.ds(i * 16, 16)] = zeros16

                @pl.when(i < _NB3 // 16)
                def _():
                    hist3[pl.ds(i * 16, 16)] = zeros16

            return 0

        lax.fori_loop(0, _NB1 // 128, _zbody, 0)

        pltpu.make_async_copy(x_hbm.at[row], buf.at[pl.ds(dbase, _N)],
                              dma_sem).wait()

        @pl.when(r < _RPW - 1)
        def _():
            nbase = ((r + 1) & 1) * _N
            pltpu.async_copy(x_hbm.at[row + 1], buf.at[pl.ds(nbase, _N)],
                             dma_sem)


        def _p0(i32, _):
            base = dbase + i32 * 32 * 16
            vs = [buf[pl.ds(base + u * 16, 16)] for u in range(32)]
            mus = [_monotonic_u32(v) for v in vs]
            b1s = [lax.convert_element_type(mu >> 21, jnp.int32) for mu in mus]
            for u in range(32):
                buf[pl.ds(base + u * 16, 16)] = plsc.bitcast(mus[u], jnp.float32)
            for u in range(32):
                plsc.addupdate_scatter(hist1, [b1s[u]], ones)
            return 0

        lax.fori_loop(0, _NCHUNK // 32, _p0, 0)

        bkt1, above1, s1 = _suffix_select(hist1, _NB1, tots, ts, jnp.int32(_K))
        k2 = jnp.int32(_K) - above1
        b1u = lax.convert_element_type(bkt1, jnp.uint32)
        csub = above1 + s1

        def _refine(mu_of, pos_of, nchunks, valid_of):
            ng = (nchunks + 3) >> 2

            def _p2(g, _):
                base = g * 4
                mus = [mu_of(base + u) for u in range(4)]
                ms = [valid_of(base + u) & ((mus[u] >> 21) == b1u)
                      for u in range(4)]
                b2s = [lax.convert_element_type(
                    (mu >> 10) & jnp.uint32(0x7FF), jnp.int32) for mu in mus]
                for u in range(4):
                    plsc.addupdate_scatter(hist2, [b2s[u]], ones, mask=ms[u])
                return 0

            lax.fori_loop(0, ng, _p2, 0)
            bkt2, above2, _ = _suffix_select(hist2, _NB2, tots, ts, k2)
            k3 = k2 - above2
            pre2 = (b1u << 11) | lax.convert_element_type(bkt2, jnp.uint32)

            def _p3(g, _):
                base = g * 4
                mus = [mu_of(base + u) for u in range(4)]
                ms = [valid_of(base + u) & ((mus[u] >> 10) == pre2)
                      for u in range(4)]
                b3s = [lax.convert_element_type(mu & jnp.uint32(0x3FF),
                                                jnp.int32) for mu in mus]
                for u in range(4):
                    plsc.addupdate_scatter(hist3, [b3s[u]], ones, mask=ms[u])
                return 0

            lax.fori_loop(0, ng, _p3, 0)
            bkt3, above3, _ = _suffix_select(hist3, _NB3, tots, ts, k3)
            need = k3 - above3
            c_gt = jnp.int32(_K) - need
            t_u = (pre2 << 10) | lax.convert_element_type(bkt3, jnp.uint32)
            t_splat = jnp.broadcast_to(t_u, (16,))

            def _cbody(g, carry):
                off_gt, off_eq = carry
                base = g * 4
                mus = [mu_of(base + u) for u in range(4)]
                poss = [pos_of(base + u) for u in range(4)]
                m_gts = [valid_of(base + u) & (mus[u] > t_splat)
                         for u in range(4)]
                m_eqs = [valid_of(base + u) & (mus[u] == t_splat)
                         for u in range(4)]
                cs_gts = [plsc.cumsum(ones, mask=m) for m in m_gts]
                cs_eqs = [plsc.cumsum(ones, mask=m) for m in m_eqs]
                n_gts = [plsc.all_reduce_population_count(m) for m in m_gts]
                n_eqs = [plsc.all_reduce_population_count(m) for m in m_eqs]
                for u in range(4):
                    p_gt = off_gt + cs_gts[u] - 1
                    p_eq = off_eq + cs_eqs[u] - 1
                    plsc.store_scatter(gt_key, [p_gt],
                                       plsc.bitcast(mus[u], jnp.int32),
                                       mask=m_gts[u])
                    plsc.store_scatter(gt_idx, [p_gt], poss[u], mask=m_gts[u])
                    plsc.store_scatter(eq_idx, [p_eq], poss[u],
                                       mask=m_eqs[u] & (p_eq < need))
                    off_gt = off_gt + n_gts[u]
                    off_eq = off_eq + n_eqs[u]
                return off_gt, off_eq

            lax.fori_loop(0, ng, _cbody, (zeros16, zeros16))
            return t_splat, need, c_gt

        def _main_path(_):
            def _collect(i32, off):
                base = i32 * 32
                mus = [plsc.bitcast(buf[pl.ds(dbase + (base + u) * 16, 16)],
                                    jnp.uint32) for u in range(32)]
                ms = [(mu >> 21) >= b1u for mu in mus]
                css = [plsc.cumsum(ones, mask=m) for m in ms]
                cnts = [plsc.all_reduce_population_count(m) for m in ms]
                for u in range(32):
                    p = off + css[u] - 1
                    plsc.store_scatter(sub_key, [p],
                                       plsc.bitcast(mus[u], jnp.int32),
                                       mask=ms[u])
                    plsc.store_scatter(sub_pos, [p], (base + u) * 16 + lane,
                                       mask=ms[u])
                    off = off + cnts[u]
                return off

            lax.fori_loop(0, _NCHUNK // 32, _collect, zeros16)

            csub_splat = jnp.broadcast_to(csub, (16,))
            nsub = (csub + 15) >> 4

            def pos_of(i):
                return sub_pos[pl.ds(i * 16, 16)]

            def mu_of(i):
                return plsc.bitcast(sub_key[pl.ds(i * 16, 16)], jnp.uint32)

            def valid_of(i):
                return (i * 16 + lane) < csub_splat

            return _refine(mu_of, pos_of, nsub, valid_of)

        def _fallback(_):
            def mu_of(i):
                return plsc.bitcast(buf[pl.ds(dbase + i * 16, 16)], jnp.uint32)

            def pos_of(i):
                return i * 16 + lane

            def valid_of(i):
                return lane >= 0

            return _refine(mu_of, pos_of, _NCHUNK, valid_of)

        t_splat, need, c_gt = lax.cond(csub <= _CAP, _main_path, _fallback, 0)

        js = [t * 16 + lane for t in range(_K // 16)]
        fgs = [j < c_gt for j in js]
        jgs = [jnp.where(fgs[t], js[t], 0) for t in range(_K // 16)]
        jes = [jnp.maximum(js[t] - c_gt, 0) for t in range(_K // 16)]
        kks = [plsc.load_gather(gt_key, [jg]) for jg in jgs]
        gis = [plsc.load_gather(gt_idx, [jg]) for jg in jgs]
        eis = [plsc.load_gather(eq_idx, [je]) for je in jes]
        ks = [jnp.where(fgs[t], plsc.bitcast(kks[t], jnp.uint32), t_splat)
              for t in range(_K // 16)]
        ps = js
        for t in range(_K // 16):
            cand_idx[pl.ds(t * 16, 16)] = jnp.where(fgs[t], gis[t], eis[t])

        ks, ps = _sort16(ks, ps, descending=True)
        for t in range(_K // 16):
            cand_key[pl.ds(t * 16, 16)] = plsc.bitcast(ks[t], jnp.int32)

        shs = [plsc.load_gather(cand_key,
                                [jnp.maximum(t * 16 + lane - 1, 0)])
               for t in range(_K // 16)]
        sts = []
        for t in range(_K // 16):
            qidx = t * 16 + lane
            d = ks[t] != plsc.bitcast(shs[t], jnp.uint32)
            if t == 0:
                d = d | (lane == 0)
            sts.append(jnp.where(d, qidx, 0))
        comp = []
        carry = jnp.int32(0)
        for t in range(_K // 16):
            rs = jnp.maximum(plsc.cummax(sts[t]), carry)
            comp.append((rs << 8) + ps[t])
            carry = jnp.maximum(carry, jnp.max(sts[t]))
        _, ps2 = _sort16(comp, ps, descending=False)

        for t in range(_K // 16):
            outrow[pl.ds(t * 16, 16)] = plsc.load_gather(cand_idx, [ps2[t]])

        pltpu.sync_copy(outrow, out_hbm.at[row])
        return 0

    lax.fori_loop(0, _RPW, row_body, 0)


@jax.jit
def kernel(x):
    mesh = plsc.VectorSubcoreMesh(core_axis_name="c", subcore_axis_name="s")
    f = functools.partial(
        pl.kernel,
        mesh=mesh,
        compiler_params=pltpu.CompilerParams(needs_layout_passes=False),
        out_type=jax.ShapeDtypeStruct((_R, _K), jnp.int32),
        scratch_types=[
            pltpu.VMEM((2 * _N,), jnp.float32),
            pltpu.VMEM((_NB1,), jnp.int32),
            pltpu.VMEM((_NB2,), jnp.int32),
            pltpu.VMEM((_NB3,), jnp.int32),
            pltpu.VMEM((256,), jnp.int32),
            pltpu.VMEM((256,), jnp.int32),
            pltpu.VMEM((_CAP + 16,), jnp.int32),
            pltpu.VMEM((_CAP + 16,), jnp.int32),
            pltpu.VMEM((512,), jnp.int32),
            pltpu.VMEM((512,), jnp.int32),
            pltpu.VMEM((512,), jnp.int32),
            pltpu.VMEM((_K,), jnp.int32),
            pltpu.VMEM((_K,), jnp.int32),
            pltpu.VMEM((_K,), jnp.int32),
            pltpu.SemaphoreType.DMA,
        ],
    )(_topk_body)
    return f(x)
